# baseline (speedup 1.0000x reference)
import functools

import numpy as np
import jax
import jax.numpy as jnp
from jax import lax
from jax.experimental import pallas as pl
from jax.experimental.pallas import tpu as pltpu

N = 9
CS = 128
CA = 32


def _body(xs_hbm, xa_hbm, wv_hbm, gam_ref, bet_ref, gw_ref, gb_ref,
          fcw_ref, fcb_ref, ei_ref, o1_ref, o2_ref, xs_v, xa_v, wv_v, sem):
    f32 = jnp.float32
    cps = [pltpu.make_async_copy(s, d, sem) for s, d in
           ((xs_hbm, xs_v), (xa_hbm, xa_v), (wv_hbm, wv_v))]
    for cp in cps:
        cp.start()
    for cp in cps:
        cp.wait()
    xs = xs_v[...].reshape(CS, N)
    xa = xa_v[...].reshape(CA, N)
    wv = wv_v[...].reshape(CS + CA, N)
    dn = (((0,), (0,)), ((), ()))
    p_qt = (lax.dot_general(xs, wv[:CS], dn, preferred_element_type=f32) +
            lax.dot_general(xa, wv[CS:], dn, preferred_element_type=f32))
    qi = lax.broadcasted_iota(jnp.int32, (N, N), 0)
    ti = lax.broadcasted_iota(jnp.int32, (N, N), 1)
    lane1 = lax.broadcasted_iota(jnp.int32, (N,), 0)
    h = jnp.zeros((N,), f32)
    for p in range(N):
        dy = qi // 3 - p // 3
        dx = qi % 3 - p % 3
        valid = ((jnp.abs(dy) <= 1) & (jnp.abs(dx) <= 1) &
                 (ti == (dy + 1) * 3 + (dx + 1)))
        h = h + ((lane1 == p).astype(f32) *
                 jnp.sum(jnp.where(valid, p_qt, 0.0)))

    mean = jnp.mean(h)
    var = jnp.mean((h - mean) ** 2)
    hn = (h - mean) * lax.rsqrt(var + 1e-5) * gam_ref[0] + bet_ref[0]
    feats = jnp.maximum(hn, 0.0)

    iota9 = lax.broadcasted_iota(jnp.int32, (N, 81), 0)
    son = (iota9 == ei_ref[0][None, :]).astype(f32)
    don = (iota9 == ei_ref[1][None, :]).astype(f32)
    m_ds = lax.dot_general(don, son, (((1,), (1,)), ((), ())),
                           preferred_element_type=f32)
    agg = jnp.sum(m_ds * feats[None, :], axis=1)

    h2 = jnp.maximum(jnp.sum(gw_ref[...] * agg[None, :], axis=1) +
                     gb_ref[...], 0.0)
    r = jnp.tanh(jnp.sum(fcw_ref[0] * h2) + fcb_ref[0])

    o1_ref[...] = h2.reshape(1, 1, 3, 3)
    o2_ref[...] = jnp.full((1,), r, f32)


@functools.partial(
    pl.pallas_call,
    out_shape=(jax.ShapeDtypeStruct((1, 1, 3, 3), jnp.float32),
               jax.ShapeDtypeStruct((1,), jnp.float32)),
    in_specs=[pl.BlockSpec(memory_space=pl.ANY)] * 3 +
             [pl.BlockSpec(memory_space=pltpu.MemorySpace.VMEM)] * 7,
    scratch_shapes=[
        pltpu.VMEM((1, CS, 3, 3), jnp.float32),
        pltpu.VMEM((1, CA, 3, 3), jnp.float32),
        pltpu.VMEM((1, CS + CA, 3, 3), jnp.float32),
        pltpu.SemaphoreType.DMA,
    ],
)
def _dynamics_tc(xs, xa, wv, gam, bet, gw, gb, fcw, fcb, ei, o1, o2,
                 xs_v, xa_v, wv_v, sem):
    _body(xs, xa, wv, gam, bet, gw, gb, fcw, fcb, ei, o1, o2,
          xs_v, xa_v, wv_v, sem)


def kernel(state, action, conv_w, bn_gamma, bn_beta, gcn_w, gcn_b, fc_w, fc_b,
           edge_index):
    return _dynamics_tc(state, action, conv_w, bn_gamma, bn_beta,
                        gcn_w, gcn_b, fc_w, fc_b, edge_index)


# speedup vs baseline: 1.1555x; 1.1555x over previous
import functools

import numpy as np
import jax
import jax.numpy as jnp
from jax import lax
from jax.experimental import pallas as pl
from jax.experimental.pallas import tpu as pltpu

N = 9
CS = 128
CA = 32


def _body(xc_ref, gam_ref, bet_ref, gw_ref, gb_ref,
          fcw_ref, fcb_ref, ei_ref, o1_ref, o2_ref):
    f32 = jnp.float32
    C = CS + CA
    xc = xc_ref[...].reshape(2 * C, N)
    dn = (((0,), (0,)), ((), ()))
    p_qt = lax.dot_general(xc[:C], xc[C:], dn, preferred_element_type=f32)
    qi = lax.broadcasted_iota(jnp.int32, (N, N), 0)
    ti = lax.broadcasted_iota(jnp.int32, (N, N), 1)
    lane1 = lax.broadcasted_iota(jnp.int32, (N,), 0)
    h = jnp.zeros((N,), f32)
    for p in range(N):
        dy = qi // 3 - p // 3
        dx = qi % 3 - p % 3
        valid = ((jnp.abs(dy) <= 1) & (jnp.abs(dx) <= 1) &
                 (ti == (dy + 1) * 3 + (dx + 1)))
        h = h + ((lane1 == p).astype(f32) *
                 jnp.sum(jnp.where(valid, p_qt, 0.0)))

    mean = jnp.mean(h)
    var = jnp.mean((h - mean) ** 2)
    hn = (h - mean) * lax.rsqrt(var + 1e-5) * gam_ref[0] + bet_ref[0]
    feats = jnp.maximum(hn, 0.0)

    iota9 = lax.broadcasted_iota(jnp.int32, (N, 81), 0)
    son = (iota9 == ei_ref[0][None, :]).astype(f32)
    don = (iota9 == ei_ref[1][None, :]).astype(f32)
    m_ds = lax.dot_general(don, son, (((1,), (1,)), ((), ())),
                           preferred_element_type=f32)
    agg = jnp.sum(m_ds * feats[None, :], axis=1)

    h2 = jnp.maximum(jnp.sum(gw_ref[...] * agg[None, :], axis=1) +
                     gb_ref[...], 0.0)
    r = jnp.tanh(jnp.sum(fcw_ref[0] * h2) + fcb_ref[0])

    o1_ref[...] = h2.reshape(1, 1, 3, 3)
    o2_ref[...] = jnp.full((1,), r, f32)


@functools.partial(
    pl.pallas_call,
    out_shape=(jax.ShapeDtypeStruct((1, 1, 3, 3), jnp.float32),
               jax.ShapeDtypeStruct((1,), jnp.float32)),
)
def _dynamics_tc(xc, gam, bet, gw, gb, fcw, fcb, ei, o1, o2):
    _body(xc, gam, bet, gw, gb, fcw, fcb, ei, o1, o2)


def kernel(state, action, conv_w, bn_gamma, bn_beta, gcn_w, gcn_b, fc_w, fc_b,
           edge_index):
    xc = jnp.concatenate([state, action, conv_w], axis=1)
    return _dynamics_tc(xc, bn_gamma, bn_beta,
                        gcn_w, gcn_b, fc_w, fc_b, edge_index)
